# baseline (speedup 1.0000x reference)
import functools

import jax
import jax.numpy as jnp
from jax import lax
from jax.experimental import pallas as pl
from jax.experimental.pallas import tpu as pltpu
from jax.experimental.pallas import tpu_sc as plsc

N = 2048
E = 65536
NTYPE = 4
NMAT = 4
NC, NS, L = 2, 16, 16
ROWS_PER_PASS = 512
NPASS = N // (ROWS_PER_PASS * NC)
CHUNK = E // NS
SH_WORDS = ROWS_PER_PASS * N
SLICE = SH_WORDS // NS
ZB = 8192
SCAT = 128


def _softmax_body(c1_ref, c2_ref, f1_ref, f2_ref, w_ref):
    x1 = c1_ref[...]
    x2 = c2_ref[...]
    e1 = jnp.exp(x1 - jnp.max(x1, axis=1, keepdims=True))
    f1 = e1 / jnp.sum(e1, axis=1, keepdims=True)
    e2 = jnp.exp(x2 - jnp.max(x2, axis=1, keepdims=True))
    f2 = e2 / jnp.sum(e2, axis=1, keepdims=True)
    f1_ref[...] = f1
    f2_ref[...] = f2
    w_ref[...] = jnp.concatenate([f1, f2], axis=0)


_softmax_call = pl.pallas_call(
    _softmax_body,
    out_shape=(
        jax.ShapeDtypeStruct((2, NTYPE), jnp.float32),
        jax.ShapeDtypeStruct((2, NTYPE), jnp.float32),
        jax.ShapeDtypeStruct((NMAT, NTYPE), jnp.float32),
    ),
)


@functools.partial(
    pl.kernel,
    out_type=jax.ShapeDtypeStruct((NMAT, N * N), jnp.float32),
    mesh=plsc.VectorSubcoreMesh(core_axis_name="c", subcore_axis_name="s"),
    scratch_types=[
        pltpu.VMEM((NMAT * NTYPE,), jnp.float32),
        pltpu.VMEM((NTYPE * CHUNK,), jnp.int32),
        pltpu.VMEM((NTYPE * CHUNK,), jnp.int32),
        pltpu.VMEM((NTYPE * CHUNK,), jnp.float32),
        pltpu.VMEM((1, SCAT), jnp.int32),
        pltpu.VMEM((1, SCAT), jnp.float32),
        pltpu.VMEM((ZB,), jnp.float32),
        pltpu.VMEM_SHARED((SH_WORDS,), jnp.float32),
    ],
)
def _sc_scatter(w_hbm, srcs_hbm, dsts_hbm, vals_hbm, ab_hbm,
                w_v, src_v, dst_v, val_v, idx_b, sval_b, zbuf, shared):
    core = lax.axis_index("c")
    sub = lax.axis_index("s")

    pltpu.sync_copy(w_hbm, w_v)
    w16 = w_v[pl.ds(0, L)]
    off = sub * CHUNK
    for j in range(NTYPE):
        pltpu.sync_copy(srcs_hbm.at[j].at[pl.ds(off, CHUNK)],
                        src_v.at[pl.ds(j * CHUNK, CHUNK)])
        pltpu.sync_copy(dsts_hbm.at[j].at[pl.ds(off, CHUNK)],
                        dst_v.at[pl.ds(j * CHUNK, CHUNK)])
        pltpu.sync_copy(vals_hbm.at[j].at[pl.ds(off, CHUNK)],
                        val_v.at[pl.ds(j * CHUNK, CHUNK)])

    def _zfill(i, carry):
        zbuf[pl.ds(i * L, L)] = jnp.zeros((L,), jnp.float32)
        return carry

    lax.fori_loop(0, ZB // L, _zfill, 0)

    for c in range(NMAT):
        for p in range(NPASS):
            def _zero(t, carry):
                pltpu.sync_copy(zbuf, shared.at[pl.ds(sub * SLICE + t * ZB, ZB)])
                return carry

            lax.fori_loop(0, SLICE // ZB, _zero, 0)
            plsc.subcore_barrier()

            row_lo = (p * NC + core) * ROWS_PER_PASS
            for j in range(NTYPE):
                w = w16[c * NTYPE + j]

                def _chunk(t, carry):
                    for k in range(SCAT // L):
                        o = pl.multiple_of(j * CHUNK + t * SCAT + k * L, L)
                        s16 = src_v[pl.ds(o, L)]
                        d16 = dst_v[pl.ds(o, L)]
                        v16 = val_v[pl.ds(o, L)]
                        rel = s16 - row_lo
                        m = (rel >= 0) & (rel < ROWS_PER_PASS)
                        idx_b[0, pl.ds(k * L, L)] = jnp.where(m, rel * N + d16, 0)
                        sval_b[0, pl.ds(k * L, L)] = jnp.where(m, v16 * w, 0.0)
                    pltpu.sync_copy(sval_b.at[0], shared.at[idx_b.at[0]], add=True)
                    return carry

                lax.fori_loop(0, CHUNK // SCAT, _chunk, 0)
            plsc.subcore_barrier()

            dst0 = (p * NC + core) * SH_WORDS + sub * SLICE
            pltpu.sync_copy(shared.at[pl.ds(sub * SLICE, SLICE)],
                            ab_hbm.at[c].at[pl.ds(dst0, SLICE)])
            plsc.subcore_barrier()


_BM = 512
_BN = 512


def _mm_body(a_ref, b_ref, h_ref):
    h_ref[0] = jnp.dot(a_ref[0], b_ref[0],
                       preferred_element_type=jnp.float32,
                       precision=lax.Precision.HIGHEST)


_mm_call = pl.pallas_call(
    _mm_body,
    grid=(2, N // _BM, N // _BN),
    in_specs=[
        pl.BlockSpec((1, _BM, N), lambda i, mi, ni: (i, mi, 0)),
        pl.BlockSpec((1, N, _BN), lambda i, mi, ni: (i + 2, 0, ni)),
    ],
    out_specs=pl.BlockSpec((1, _BM, _BN), lambda i, mi, ni: (i, mi, ni)),
    out_shape=jax.ShapeDtypeStruct((2, N, N), jnp.float32),
)


def kernel(edge_index_0, edge_index_1, edge_index_2, edge_index_3,
           edge_value_0, edge_value_1, edge_value_2, edge_value_3,
           conv1_weight, conv2_weight):
    f1, f2, w = _softmax_call(conv1_weight, conv2_weight)
    srcs = jnp.stack([edge_index_0[0], edge_index_1[0],
                      edge_index_2[0], edge_index_3[0]])
    dsts = jnp.stack([edge_index_0[1], edge_index_1[1],
                      edge_index_2[1], edge_index_3[1]])
    vals = jnp.stack([edge_value_0, edge_value_1, edge_value_2, edge_value_3])
    ab = _sc_scatter(w.reshape(NMAT * NTYPE), srcs, dsts, vals).reshape(NMAT, N, N)
    h = _mm_call(ab, ab)
    return (h, lax.stop_gradient(f1), lax.stop_gradient(f2))
